# baseline (speedup 1.0000x reference)
import functools

import jax
import jax.numpy as jnp
from jax import lax
from jax.experimental import pallas as pl
from jax.experimental.pallas import tpu as pltpu
from jax.experimental.pallas import tpu_sc as plsc

DIM = 128
CHUNK = 128


@functools.lru_cache(maxsize=None)
def _make_gather(B: int, E: int, R: int):
    info = plsc.get_sparse_core_info()
    NC, NS = info.num_cores, info.num_subcores
    NW = NC * NS
    assert B % (8 * NW) == 0
    b_per_w = B // NW
    assert b_per_w % CHUNK == 0
    n_chunks = b_per_w // CHUNK

    mesh = plsc.VectorSubcoreMesh(core_axis_name="c", subcore_axis_name="s")

    @functools.partial(
        pl.kernel,
        mesh=mesh,
        out_type=(
            jax.ShapeDtypeStruct((B, DIM), jnp.float32),
            jax.ShapeDtypeStruct((B, DIM), jnp.float32),
            jax.ShapeDtypeStruct((B, DIM), jnp.float32),
        ),
        scratch_types=[
            pltpu.VMEM((CHUNK,), jnp.int32),
            pltpu.VMEM((CHUNK,), jnp.int32),
            pltpu.VMEM((CHUNK, DIM), jnp.float32),
            pltpu.VMEM((CHUNK, DIM), jnp.float32),
            pltpu.SemaphoreType.DMA,
            pltpu.SemaphoreType.DMA,
            pltpu.SemaphoreType.DMA,
            pltpu.SemaphoreType.DMA,
        ],
    )
    def gather_k(ent_hbm, rel_hbm, hidx_hbm, tidx_hbm, ridx_hbm,
                 h_out, t_out, r_out, idx_v0, idx_v1, rows_v0, rows_v1,
                 gsem0, gsem1, wsem0, wsem1):
        wid = lax.axis_index("s") * NC + lax.axis_index("c")
        base = wid * b_per_w
        idx_v = (idx_v0, idx_v1)
        rows_v = (rows_v0, rows_v1)
        gsem = (gsem0, gsem1)
        wsem = (wsem0, wsem1)
        streams = (
            (ent_hbm, hidx_hbm, h_out),
            (ent_hbm, tidx_hbm, t_out),
            (rel_hbm, ridx_hbm, r_out),
        )
        jobs = [(table, idx_hbm, out_hbm, base + c * CHUNK)
                for table, idx_hbm, out_hbm in streams
                for c in range(n_chunks)]
        n = len(jobs)
        gh = [None] * n
        wh = [None] * n
        for j, (table, idx_hbm, out_hbm, off) in enumerate(jobs):
            b = j & 1
            if j >= 2:
                wh[j - 2].wait()
            pltpu.sync_copy(idx_hbm.at[pl.ds(off, CHUNK)], idx_v[b])
            gh[j] = pltpu.async_copy(table.at[idx_v[b]], rows_v[b], gsem[b])
            if j >= 1:
                pb = (j - 1) & 1
                _, _, pout, poff = jobs[j - 1]
                gh[j - 1].wait()
                wh[j - 1] = pltpu.async_copy(
                    rows_v[pb], pout.at[pl.ds(poff, CHUNK)], wsem[pb])
        gh[n - 1].wait()
        _, _, pout, poff = jobs[n - 1]
        wh[n - 1] = pltpu.async_copy(
            rows_v[(n - 1) & 1], pout.at[pl.ds(poff, CHUNK)], wsem[(n - 1) & 1])
        wh[n - 2].wait()
        wh[n - 1].wait()

    return gather_k


def _mlp_body(carry_ref, h_ref, t_ref, r_ref, W1_ref, b1_ref, W2_ref, b2_ref,
              Wp_ref, bp_ref, out_ref):
    del carry_ref
    x1 = h_ref[...]
    x2 = r_ref[...] * t_ref[...]
    ss = (jnp.sum(x1 * x1, axis=1, keepdims=True)
          + jnp.sum(x2 * x2, axis=1, keepdims=True))
    inv = 1.0 / jnp.maximum(jnp.sqrt(ss), 1e-12)
    x = jnp.concatenate([x1 * inv, x2 * inv], axis=1).astype(jnp.bfloat16)
    y = lax.dot_general(x, W1_ref[...], (((1,), (0,)), ((), ())),
                        preferred_element_type=jnp.float32) + b1_ref[...]
    y = jnp.maximum(y, 0.0).astype(jnp.bfloat16)
    y = lax.dot_general(y, W2_ref[...], (((1,), (0,)), ((), ())),
                        preferred_element_type=jnp.float32) + b2_ref[...]
    y = jnp.maximum(y, 0.0)
    s = jnp.sum(y * Wp_ref[...], axis=1, keepdims=True) + bp_ref[...]
    out_ref[...] = jax.nn.sigmoid(s)


@functools.lru_cache(maxsize=None)
def _make_mlp(B: int, H: int, blk: int, base_blk: int):
    grid = (H // blk,)
    full = lambda i: (0, 0)
    return pl.pallas_call(
        _mlp_body,
        grid=grid,
        in_specs=[
            pl.BlockSpec(memory_space=pl.ANY),
            pl.BlockSpec((blk, DIM), lambda i: (i, 0)),
            pl.BlockSpec((blk, DIM), lambda i: (i, 0)),
            pl.BlockSpec((blk, DIM), lambda i: (i, 0)),
            pl.BlockSpec((2 * DIM, 512), full),
            pl.BlockSpec((1, 512), full),
            pl.BlockSpec((512, 256), full),
            pl.BlockSpec((1, 256), full),
            pl.BlockSpec((1, 256), full),
            pl.BlockSpec((1, 1), full),
        ],
        out_specs=pl.BlockSpec((blk, 1), lambda i: (base_blk + i, 0)),
        out_shape=jax.ShapeDtypeStruct((B, 1), jnp.float32),
        input_output_aliases={0: 0},
    )


def kernel(data, eval, cf_train, ent_emb, rel_emb, W1, b1, W2, b2, Wp, bp):
    B = data.shape[0]
    hidx = data[:, 0]
    tidx = data[:, 1]
    ridx = data[:, 2]
    W1b = W1.T.astype(jnp.bfloat16)
    W2b = W2.T.astype(jnp.bfloat16)
    b1r = b1.reshape(1, -1)
    b2r = b2.reshape(1, -1)
    bpr = bp.reshape(1, 1)
    H = B // 2
    gather = _make_gather(H, ent_emb.shape[0], rel_emb.shape[0])
    blk = 4096
    out = jnp.zeros((B, 1), jnp.float32)
    for lo in (0, H):
        h, t, r = gather(ent_emb, rel_emb,
                         lax.slice(hidx, (lo,), (lo + H,)),
                         lax.slice(tidx, (lo,), (lo + H,)),
                         lax.slice(ridx, (lo,), (lo + H,)))
        out = _make_mlp(B, H, blk, lo // blk)(
            out, h, t, r, W1b, b1r, W2b, b2r, Wp, bpr)
    return out


# speedup vs baseline: 1.0969x; 1.0969x over previous
import functools

import jax
import jax.numpy as jnp
from jax import lax
from jax.experimental import pallas as pl
from jax.experimental.pallas import tpu as pltpu
from jax.experimental.pallas import tpu_sc as plsc

DIM = 128
CHUNK = 128


@functools.lru_cache(maxsize=None)
def _make_gather(B: int, E: int, R: int):
    info = plsc.get_sparse_core_info()
    NC, NS = info.num_cores, info.num_subcores
    NW = NC * NS
    assert B % (8 * NW) == 0
    b_per_w = B // NW
    assert b_per_w % CHUNK == 0
    n_chunks = b_per_w // CHUNK

    mesh = plsc.VectorSubcoreMesh(core_axis_name="c", subcore_axis_name="s")

    @functools.partial(
        pl.kernel,
        mesh=mesh,
        out_type=(
            jax.ShapeDtypeStruct((B, DIM), jnp.float32),
            jax.ShapeDtypeStruct((B, DIM), jnp.float32),
        ),
        scratch_types=[
            pltpu.VMEM((CHUNK,), jnp.int32),
            pltpu.VMEM((CHUNK,), jnp.int32),
            pltpu.VMEM((CHUNK,), jnp.int32),
            pltpu.VMEM((CHUNK,), jnp.int32),
            pltpu.VMEM((CHUNK,), jnp.int32),
            pltpu.VMEM((CHUNK,), jnp.int32),
            pltpu.VMEM((CHUNK, DIM), jnp.float32),
            pltpu.VMEM((CHUNK, DIM), jnp.float32),
            pltpu.VMEM((CHUNK, DIM), jnp.float32),
            pltpu.VMEM((CHUNK, DIM), jnp.float32),
            pltpu.VMEM((CHUNK, DIM), jnp.float32),
            pltpu.VMEM((CHUNK, DIM), jnp.float32),
            pltpu.SemaphoreType.DMA,
            pltpu.SemaphoreType.DMA,
            pltpu.SemaphoreType.DMA,
            pltpu.SemaphoreType.DMA,
            pltpu.SemaphoreType.DMA,
            pltpu.SemaphoreType.DMA,
            pltpu.SemaphoreType.DMA,
            pltpu.SemaphoreType.DMA,
        ],
    )
    def gather_k(ent_hbm, rel_hbm, hidx_hbm, tidx_hbm, ridx_hbm,
                 h_out, p_out,
                 ih0, ih1, it0, it1, ir0, ir1,
                 bh0, bh1, bt0, bt1, br0, br1,
                 gsh0, gsh1, gst0, gst1, gsr0, gsr1, wsh, wsp):
        wid = lax.axis_index("s") * NC + lax.axis_index("c")
        base = wid * b_per_w
        ih = (ih0, ih1)
        it = (it0, it1)
        ir = (ir0, ir1)
        bh = (bh0, bh1)
        bt = (bt0, bt1)
        br = (br0, br1)
        gsh = (gsh0, gsh1)
        gst = (gst0, gst1)
        gsr = (gsr0, gsr1)

        def mul_into(tb, rb):
            def body(row, _):
                for g in range(DIM // 16):
                    sl = pl.ds(g * 16, 16)
                    tb[row, sl] = tb[row, sl] * rb[row, sl]
                return 0
            lax.fori_loop(0, CHUNK, body, 0)

        def start_unit(c):
            b = c & 1
            off = base + c * CHUNK
            pltpu.sync_copy(hidx_hbm.at[pl.ds(off, CHUNK)], ih[b])
            pltpu.sync_copy(tidx_hbm.at[pl.ds(off, CHUNK)], it[b])
            pltpu.sync_copy(ridx_hbm.at[pl.ds(off, CHUNK)], ir[b])
            return (pltpu.async_copy(ent_hbm.at[ih[b]], bh[b], gsh[b]),
                    pltpu.async_copy(ent_hbm.at[it[b]], bt[b], gst[b]),
                    pltpu.async_copy(rel_hbm.at[ir[b]], br[b], gsr[b]))

        def finish_unit(c, handles):
            b = c & 1
            off = base + c * CHUNK
            hh, ht, hr = handles
            hh.wait()
            wh = pltpu.async_copy(bh[b], h_out.at[pl.ds(off, CHUNK)], wsh)
            ht.wait()
            hr.wait()
            mul_into(bt[b], br[b])
            wp = pltpu.async_copy(bt[b], p_out.at[pl.ds(off, CHUNK)], wsp)
            return wh, wp

        pend = [None] * n_chunks
        writes = [None] * n_chunks
        for c in range(n_chunks):
            pend[c] = start_unit(c)
            if c >= 1:
                writes[c - 1] = finish_unit(c - 1, pend[c - 1])
            if c >= 2:
                for w in writes[c - 2]:
                    w.wait()
        writes[n_chunks - 1] = finish_unit(n_chunks - 1, pend[n_chunks - 1])
        if n_chunks >= 2:
            for w in writes[n_chunks - 2]:
                w.wait()
        for w in writes[n_chunks - 1]:
            w.wait()

    return gather_k


def _mlp_body(h_ref, p_ref, W1_ref, b1_ref, W2_ref, b2_ref,
              Wp_ref, bp_ref, out_ref):
    x1 = h_ref[...]
    x2 = p_ref[...]
    ss = (jnp.sum(x1 * x1, axis=1, keepdims=True)
          + jnp.sum(x2 * x2, axis=1, keepdims=True))
    inv = 1.0 / jnp.maximum(jnp.sqrt(ss), 1e-12)
    x = jnp.concatenate([x1 * inv, x2 * inv], axis=1).astype(jnp.bfloat16)
    y = lax.dot_general(x, W1_ref[...], (((1,), (0,)), ((), ())),
                        preferred_element_type=jnp.float32) + b1_ref[...]
    y = jnp.maximum(y, 0.0).astype(jnp.bfloat16)
    y = lax.dot_general(y, W2_ref[...], (((1,), (0,)), ((), ())),
                        preferred_element_type=jnp.float32) + b2_ref[...]
    y = jnp.maximum(y, 0.0)
    s = jnp.sum(y * Wp_ref[...], axis=1, keepdims=True) + bp_ref[...]
    out_ref[...] = jax.nn.sigmoid(s)


@functools.lru_cache(maxsize=None)
def _make_mlp(H: int, blk: int):
    grid = (H // blk,)
    full = lambda i: (0, 0)
    return pl.pallas_call(
        _mlp_body,
        grid=grid,
        in_specs=[
            pl.BlockSpec((blk, DIM), lambda i: (i, 0)),
            pl.BlockSpec((blk, DIM), lambda i: (i, 0)),
            pl.BlockSpec((2 * DIM, 512), full),
            pl.BlockSpec((1, 512), full),
            pl.BlockSpec((512, 256), full),
            pl.BlockSpec((1, 256), full),
            pl.BlockSpec((1, 256), full),
            pl.BlockSpec((1, 1), full),
        ],
        out_specs=pl.BlockSpec((blk, 1), lambda i: (i, 0)),
        out_shape=jax.ShapeDtypeStruct((H, 1), jnp.float32),
    )


def kernel(data, eval, cf_train, ent_emb, rel_emb, W1, b1, W2, b2, Wp, bp):
    B = data.shape[0]
    hidx = data[:, 0]
    tidx = data[:, 1]
    ridx = data[:, 2]
    W1b = W1.T.astype(jnp.bfloat16)
    W2b = W2.T.astype(jnp.bfloat16)
    b1r = b1.reshape(1, -1)
    b2r = b2.reshape(1, -1)
    bpr = bp.reshape(1, 1)
    H = B // 2
    gather = _make_gather(H, ent_emb.shape[0], rel_emb.shape[0])
    mlp = _make_mlp(H, 4096)
    scores = []
    for lo in (0, H):
        h, p = gather(ent_emb, rel_emb,
                      lax.slice(hidx, (lo,), (lo + H,)),
                      lax.slice(tidx, (lo,), (lo + H,)),
                      lax.slice(ridx, (lo,), (lo + H,)))
        scores.append(mlp(h, p, W1b, b1r, W2b, b2r, Wp, bpr))
    return jnp.concatenate(scores, axis=0)
